# baseline (speedup 1.0000x reference)
import functools

import jax
import jax.numpy as jnp
from jax import lax
from jax.experimental import pallas as pl
from jax.experimental.pallas import tpu as pltpu
from jax.experimental.pallas import tpu_sc as plsc

D = 64
T = 200
NC = 2
NS = 16
NW = NC * NS
C = 128
LANES = 16
NBUF = 4


def _add_positions(rows_v, pos_v, stage_v, g):
    p = lax.rem(g * C, T)

    def pair_body(r2, carry):
        r = 2 * r2
        for c in range(D // LANES):
            sl = pl.ds(c * LANES, LANES)
            stage_v[r2, pl.ds(c * LANES, LANES)] = (
                rows_v[r, sl] + pos_v[p + r, sl])
            stage_v[r2, pl.ds(D + c * LANES, LANES)] = (
                rows_v[r + 1, sl] + pos_v[p + r + 1, sl])
        return carry

    lax.fori_loop(0, C // 2, pair_body, 0, unroll=4)


def _sc_lookup(flat_ids, table, pos2):
    n_flat = flat_ids.shape[0]
    per_w = n_flat // NW
    n_chunks = per_w // C

    mesh = plsc.VectorSubcoreMesh(core_axis_name="c", subcore_axis_name="s")

    @functools.partial(
        pl.kernel,
        mesh=mesh,
        compiler_params=pltpu.CompilerParams(use_tc_tiling_on_sc=False),
        out_type=jax.ShapeDtypeStruct((n_flat // 2, 2 * D), jnp.float32),
        scratch_types=[
            pltpu.VMEM((per_w,), jnp.int32),
            pltpu.VMEM((2 * T, D), jnp.float32),
        ] + [pltpu.VMEM((C, D), jnp.float32)] * NBUF
          + [pltpu.VMEM((C // 2, 2 * D), jnp.float32)] * NBUF
          + [pltpu.SemaphoreType.DMA] * (2 * NBUF),
    )
    def body(ids_hbm, table_hbm, pos2_hbm, out_hbm, idx_all, pos_v,
             r0, r1, r2, r3, t0, t1, t2, t3,
             sg0, sg1, sg2, sg3, sw0, sw1, sw2, sw3):
        rows = (r0, r1, r2, r3)
        stage = (t0, t1, t2, t3)
        sg = (sg0, sg1, sg2, sg3)
        sw = (sw0, sw1, sw2, sw3)
        wid = lax.axis_index("s") * NC + lax.axis_index("c")
        base = wid * per_w
        pltpu.sync_copy(ids_hbm.at[pl.ds(base, per_w)], idx_all)
        pltpu.sync_copy(pos2_hbm, pos_v)

        def gather_start(g, b):
            pltpu.async_copy(table_hbm.at[idx_all.at[pl.ds(g * C, C)]],
                             rows[b], sg[b])

        def gather_wait(b):
            pltpu.make_async_copy(table_hbm.at[idx_all.at[pl.ds(0, C)]],
                                  rows[b], sg[b]).wait()

        def write_start(g, b):
            pltpu.async_copy(
                stage[b],
                out_hbm.at[pl.ds((base + g * C) // 2, C // 2)], sw[b])

        def write_wait(b):
            pltpu.make_async_copy(stage[b],
                                  out_hbm.at[pl.ds(base // 2, C // 2)],
                                  sw[b]).wait()

        for b in range(NBUF):
            gather_start(b, b)

        for b in range(NBUF):
            gather_wait(b)
            _add_positions(rows[b], pos_v, stage[b], b)
            gather_start(NBUF + b, b)
            write_start(b, b)

        def main_body(i, carry):
            k = i * NBUF
            for b in range(NBUF):
                gather_wait(b)
                write_wait(b)
                _add_positions(rows[b], pos_v, stage[b], k + b)
                gather_start(k + NBUF + b, b)
                write_start(k + b, b)
            return carry

        lax.fori_loop(1, n_chunks // NBUF - 1, main_body, 0)

        k = n_chunks - NBUF
        for b in range(NBUF):
            gather_wait(b)
            write_wait(b)
            _add_positions(rows[b], pos_v, stage[b], k + b)
            write_start(k + b, b)
        for b in range(NBUF):
            write_wait(b)

    return body(flat_ids, table, pos2)


def kernel(input_ids, phoneme_table, position_table):
    b, t = input_ids.shape
    flat_ids = input_ids.reshape(-1).astype(jnp.int32)
    pos2 = jnp.concatenate([position_table, position_table], axis=0)
    out = _sc_lookup(flat_ids, phoneme_table, pos2)
    return out.reshape(b, t, D)


# speedup vs baseline: 1.0198x; 1.0198x over previous
import functools

import jax
import jax.numpy as jnp
from jax import lax
from jax.experimental import pallas as pl
from jax.experimental.pallas import tpu as pltpu
from jax.experimental.pallas import tpu_sc as plsc

D = 64
T = 200
NC = 2
NS = 16
NW = NC * NS
C = 128
NBUF = 4
G = 8


def _sc_gather(flat_ids, table):
    n_flat = flat_ids.shape[0]
    per_w = n_flat // NW
    n_chunks = per_w // C

    mesh = plsc.VectorSubcoreMesh(core_axis_name="c", subcore_axis_name="s")

    @functools.partial(
        pl.kernel,
        mesh=mesh,
        compiler_params=pltpu.CompilerParams(use_tc_tiling_on_sc=False),
        out_type=jax.ShapeDtypeStruct((n_flat, D), jnp.float32),
        scratch_types=[
            pltpu.VMEM((per_w,), jnp.int32),
        ] + [pltpu.VMEM((C, D), jnp.float32)] * NBUF
          + [pltpu.SemaphoreType.DMA] * (2 * NBUF),
    )
    def body(ids_hbm, table_hbm, out_hbm, idx_all,
             r0, r1, r2, r3, sg0, sg1, sg2, sg3, sw0, sw1, sw2, sw3):
        rows = (r0, r1, r2, r3)
        sg = (sg0, sg1, sg2, sg3)
        sw = (sw0, sw1, sw2, sw3)
        wid = lax.axis_index("s") * NC + lax.axis_index("c")
        base = wid * per_w
        pltpu.sync_copy(ids_hbm.at[pl.ds(base, per_w)], idx_all)

        def gather_start(g, b):
            pltpu.async_copy(table_hbm.at[idx_all.at[pl.ds(g * C, C)]],
                             rows[b], sg[b])

        def gather_wait(b):
            pltpu.make_async_copy(table_hbm.at[idx_all.at[pl.ds(0, C)]],
                                  rows[b], sg[b]).wait()

        def write_start(g, b):
            pltpu.async_copy(rows[b], out_hbm.at[pl.ds(base + g * C, C)],
                             sw[b])

        def write_wait(b):
            pltpu.make_async_copy(rows[b], out_hbm.at[pl.ds(base, C)],
                                  sw[b]).wait()

        for b in range(NBUF):
            gather_start(b, b)

        def main_body(i, carry):
            k = i * NBUF
            for b in range(NBUF):
                gather_wait(b)
                write_start(k + b, b)
            for b in range(NBUF):
                write_wait(b)
                gather_start(k + NBUF + b, b)
            return carry

        lax.fori_loop(0, n_chunks // NBUF - 1, main_body, 0)

        k = n_chunks - NBUF
        for b in range(NBUF):
            gather_wait(b)
            write_start(k + b, b)
        for b in range(NBUF):
            write_wait(b)

    return body(flat_ids, table)


def _tc_add(inter, pos_pair, batch):
    half = T // 2

    def tc_body(inter_ref, pos_ref, out_ref):
        x = inter_ref[...].reshape(G, half, 2 * D) + pos_ref[...][None]
        y = jnp.stack((x[:, :, :D], x[:, :, D:]), axis=2)
        out_ref[...] = y.reshape(G, T, D)

    return pl.pallas_call(
        tc_body,
        grid=(batch // G,),
        in_specs=[
            pl.BlockSpec((G * half, 2 * D), lambda i: (i, 0)),
            pl.BlockSpec((half, 2 * D), lambda i: (0, 0)),
        ],
        out_specs=pl.BlockSpec((G, T, D), lambda i: (i, 0, 0)),
        out_shape=jax.ShapeDtypeStruct((batch, T, D), jnp.float32),
    )(inter, pos_pair)


def kernel(input_ids, phoneme_table, position_table):
    b, t = input_ids.shape
    flat_ids = input_ids.reshape(-1).astype(jnp.int32)
    pos_pair = position_table.reshape(T // 2, 2 * D)
    inter = _sc_gather(flat_ids, phoneme_table)
    inter = inter.reshape(b * t // 2, 2 * D)
    return _tc_add(inter, pos_pair, b)
